# Initial kernel scaffold; baseline (speedup 1.0000x reference)
import jax
import jax.numpy as jnp
from jax.experimental import pallas as pl


def kernel(element_indices, element_fractions, element_mask, oxidation_states):
    raise NotImplementedError("write your pallas kernel here")


# speedup vs baseline: 35.6566x; 35.6566x over previous
import functools

import jax
import jax.numpy as jnp
from jax import lax
from jax.experimental import pallas as pl
from jax.experimental.pallas import tpu as pltpu
from jax.experimental.pallas import tpu_sc as plsc

_B = 16384
_L = 20
_NC = 2
_NS = 16
_NW = _NC * _NS
_LANES = 16
_ROWS_PER_W = _B // _NW
_ELEMS_PER_W = _ROWS_PER_W * _L
_TOL = 0.5
_TABLE_PAD = 128

_mesh = plsc.VectorSubcoreMesh(
    core_axis_name="c", subcore_axis_name="s",
    num_cores=_NC, num_subcores=_NS)


@functools.partial(
    pl.kernel,
    out_type=jax.ShapeDtypeStruct((_NW, 2, _LANES), jnp.float32),
    mesh=_mesh,
    compiler_params=pltpu.CompilerParams(needs_layout_passes=False),
    scratch_types=[
        pltpu.VMEM((_ELEMS_PER_W,), jnp.int32),
        pltpu.VMEM((_ELEMS_PER_W,), jnp.float32),
        pltpu.VMEM((_ELEMS_PER_W,), jnp.float32),
        pltpu.VMEM((_TABLE_PAD,), jnp.float32),
        pltpu.VMEM((_ELEMS_PER_W,), jnp.float32),
        pltpu.VMEM((2, _LANES), jnp.float32),
    ],
)
def _sc_charge_loss(idx_hbm, frac_hbm, maskf_hbm, table_hbm, out_hbm,
                    idx_v, frac_v, maskf_v, table_v, charge_v, out_v):
    wid = lax.axis_index("s") * _NC + lax.axis_index("c")
    base = wid * _ELEMS_PER_W
    pltpu.sync_copy(table_hbm, table_v)
    pltpu.sync_copy(idx_hbm.at[pl.ds(base, _ELEMS_PER_W)], idx_v)
    pltpu.sync_copy(frac_hbm.at[pl.ds(base, _ELEMS_PER_W)], frac_v)
    pltpu.sync_copy(maskf_hbm.at[pl.ds(base, _ELEMS_PER_W)], maskf_v)

    def phase1(i, carry):
        sl = pl.ds(i * _LANES, _LANES)
        idx = jnp.clip(idx_v[sl], 0, _TABLE_PAD - 1)
        ox = plsc.load_gather(table_v, [idx])
        charge_v[sl] = frac_v[sl] * maskf_v[sl] * ox
        return carry
    lax.fori_loop(0, _ELEMS_PER_W // _LANES, phase1, 0)

    iota = lax.iota(jnp.int32, _LANES)

    def phase2(j, carry):
        loss_acc, abs_acc = carry
        ebase = (j * _LANES + iota) * _L

        def inner(l, tc):
            return tc + plsc.load_gather(charge_v, [ebase + l])
        tc = lax.fori_loop(0, _L, inner, jnp.zeros((_LANES,), jnp.float32))
        a = jnp.abs(tc)
        ex = jnp.maximum(a - _TOL, 0.0)
        e2 = jnp.exp(2.0 * ex)
        t = 1.0 - 2.0 / (e2 + 1.0)
        return loss_acc + t, abs_acc + a

    zero = jnp.zeros((_LANES,), jnp.float32)
    loss_acc, abs_acc = lax.fori_loop(
        0, _ROWS_PER_W // _LANES, phase2, (zero, zero))

    out_v[0, :] = loss_acc * (1.0 / _B)
    out_v[1, :] = abs_acc * (1.0 / _B)
    pltpu.sync_copy(out_v, out_hbm.at[wid])


def kernel(element_indices, element_fractions, element_mask, oxidation_states):
    ei = element_indices.astype(jnp.int32).reshape(-1)
    ef = element_fractions.reshape(-1)
    em = element_mask.astype(jnp.float32).reshape(-1)
    table = jnp.concatenate(
        [oxidation_states,
         jnp.zeros((_TABLE_PAD - oxidation_states.shape[0],), jnp.float32)])
    partials = _sc_charge_loss(ei, ef, em, table)
    charge_balance_loss = jnp.sum(partials[:, 0, :])
    mean_charge_imbalance = jnp.sum(partials[:, 1, :])
    return (charge_balance_loss, mean_charge_imbalance)


# speedup vs baseline: 43.7234x; 1.2262x over previous
import functools

import jax
import jax.numpy as jnp
from jax import lax
from jax.experimental import pallas as pl
from jax.experimental.pallas import tpu as pltpu
from jax.experimental.pallas import tpu_sc as plsc

_B = 16384
_L = 20
_NC = 2
_NS = 16
_NW = _NC * _NS
_LANES = 16
_ROWS_PER_W = _B // _NW
_ELEMS_PER_W = _ROWS_PER_W * _L
_TOL = 0.5
_TSTRIDE = _ROWS_PER_W + 1
_UNROLL = 4

_mesh = plsc.VectorSubcoreMesh(
    core_axis_name="c", subcore_axis_name="s",
    num_cores=_NC, num_subcores=_NS)


@functools.partial(
    pl.kernel,
    out_type=jax.ShapeDtypeStruct((_NW, 2, _LANES), jnp.float32),
    mesh=_mesh,
    compiler_params=pltpu.CompilerParams(needs_layout_passes=False),
    scratch_types=[
        pltpu.VMEM((_ELEMS_PER_W,), jnp.int32),
        pltpu.VMEM((_ELEMS_PER_W,), jnp.float32),
        pltpu.VMEM((120,), jnp.float32),
        pltpu.VMEM((_L * _TSTRIDE,), jnp.float32),
        pltpu.VMEM((2, _LANES), jnp.float32),
    ],
)
def _sc_charge_loss(pw_hbm, frac_hbm, table_hbm, out_hbm,
                    pw_v, frac_v, table_v, charge_t, out_v):
    wid = lax.axis_index("s") * _NC + lax.axis_index("c")
    base = wid * _ELEMS_PER_W
    pltpu.sync_copy(table_hbm, table_v)
    pltpu.sync_copy(pw_hbm.at[pl.ds(base, _ELEMS_PER_W)], pw_v)
    pltpu.sync_copy(frac_hbm.at[pl.ds(base, _ELEMS_PER_W)], frac_v)

    iota = lax.iota(jnp.int32, _LANES)

    def phase1(i, carry):
        for u in range(_UNROLL):
            s = (i * _UNROLL + u) * _LANES
            sl = pl.ds(s, _LANES)
            w = pw_v[sl]
            idx = jnp.minimum(w & 127, 119)
            mf = (w >> 7).astype(jnp.float32)
            ox = plsc.load_gather(table_v, [idx])
            e = s + iota
            tidx = (e % _L) * _TSTRIDE + e // _L
            plsc.store_scatter(charge_t, [tidx], frac_v[sl] * mf * ox)
        return carry
    lax.fori_loop(0, _ELEMS_PER_W // (_LANES * _UNROLL), phase1, 0)

    def phase2(j, carry):
        loss_acc, abs_acc = carry
        r = j * _LANES
        tc = charge_t[pl.ds(r, _LANES)]
        for l in range(1, _L):
            tc = tc + charge_t[pl.ds(l * _TSTRIDE + r, _LANES)]
        a = jnp.abs(tc)
        ex = jnp.maximum(a - _TOL, 0.0)
        e2 = jnp.exp(2.0 * ex)
        t = 1.0 - 2.0 / (e2 + 1.0)
        return loss_acc + t, abs_acc + a

    zero = jnp.zeros((_LANES,), jnp.float32)
    loss_acc, abs_acc = lax.fori_loop(
        0, _ROWS_PER_W // _LANES, phase2, (zero, zero))

    out_v[0, :] = loss_acc * (1.0 / _B)
    out_v[1, :] = abs_acc * (1.0 / _B)
    pltpu.sync_copy(out_v, out_hbm.at[wid])


def kernel(element_indices, element_fractions, element_mask, oxidation_states):
    pw = (element_indices.astype(jnp.int32)
          | (element_mask.astype(jnp.int32) << 7)).reshape(-1)
    ef = element_fractions.reshape(-1)
    partials = _sc_charge_loss(pw, ef, oxidation_states)
    charge_balance_loss = jnp.sum(partials[:, 0, :])
    mean_charge_imbalance = jnp.sum(partials[:, 1, :])
    return (charge_balance_loss, mean_charge_imbalance)


# speedup vs baseline: 43.7384x; 1.0003x over previous
import functools

import jax
import jax.numpy as jnp
from jax import lax
from jax.experimental import pallas as pl
from jax.experimental.pallas import tpu as pltpu
from jax.experimental.pallas import tpu_sc as plsc

_B = 16384
_L = 20
_NC = 2
_NS = 16
_NW = _NC * _NS
_LANES = 16
_ROWS_PER_W = _B // _NW
_ELEMS_PER_W = _ROWS_PER_W * _L
_TOL = 0.5
_TSTRIDE = _ROWS_PER_W + 1
_UNROLL = 4

_mesh = plsc.VectorSubcoreMesh(
    core_axis_name="c", subcore_axis_name="s",
    num_cores=_NC, num_subcores=_NS)


@functools.partial(
    pl.kernel,
    out_type=jax.ShapeDtypeStruct((_NW, 2, _LANES), jnp.float32),
    mesh=_mesh,
    compiler_params=pltpu.CompilerParams(needs_layout_passes=False),
    scratch_types=[
        pltpu.VMEM((_ELEMS_PER_W,), jnp.int32),
        pltpu.VMEM((_ELEMS_PER_W,), jnp.float32),
        pltpu.VMEM((120,), jnp.float32),
        pltpu.VMEM((_L * _TSTRIDE,), jnp.float32),
        pltpu.VMEM((2, _LANES), jnp.float32),
    ],
)
def _sc_charge_loss(pw_hbm, frac_hbm, table_hbm, out_hbm,
                    pw_v, frac_v, table_v, charge_t, out_v):
    wid = lax.axis_index("s") * _NC + lax.axis_index("c")
    base = wid * _ELEMS_PER_W
    pltpu.sync_copy(table_hbm, table_v)
    pltpu.sync_copy(pw_hbm.at[pl.ds(base, _ELEMS_PER_W)], pw_v)
    pltpu.sync_copy(frac_hbm.at[pl.ds(base, _ELEMS_PER_W)], frac_v)

    iota = lax.iota(jnp.int32, _LANES)

    def phase1(i, carry):
        for u in range(_UNROLL):
            s = (i * _UNROLL + u) * _LANES
            sl = pl.ds(s, _LANES)
            w = pw_v[sl]
            idx = jnp.minimum(w & 127, 119)
            mf = (w >> 7).astype(jnp.float32)
            ox = plsc.load_gather(table_v, [idx])
            e = s + iota
            tidx = (e % _L) * _TSTRIDE + e // _L
            plsc.store_scatter(charge_t, [tidx], frac_v[sl] * mf * ox)
        return carry
    lax.fori_loop(0, _ELEMS_PER_W // (_LANES * _UNROLL), phase1, 0)

    def phase2(j, carry):
        loss_acc, abs_acc = carry
        r = j * _LANES
        tc = charge_t[pl.ds(r, _LANES)]
        for l in range(1, _L):
            tc = tc + charge_t[pl.ds(l * _TSTRIDE + r, _LANES)]
        a = jnp.abs(tc)
        ex = jnp.maximum(a - _TOL, 0.0)
        e2 = jnp.exp(2.0 * ex)
        t = 1.0 - 2.0 / (e2 + 1.0)
        return loss_acc + t, abs_acc + a

    zero = jnp.zeros((_LANES,), jnp.float32)
    loss_acc, abs_acc = lax.fori_loop(
        0, _ROWS_PER_W // _LANES, phase2, (zero, zero))

    out_v[0, :] = loss_acc * (1.0 / _B)
    out_v[1, :] = abs_acc * (1.0 / _B)
    pltpu.sync_copy(out_v, out_hbm.at[wid])


def kernel(element_indices, element_fractions, element_mask, oxidation_states):
    pw = (element_indices.reshape(-1).astype(jnp.int32)
          | (element_mask.reshape(-1).astype(jnp.int32) << 7))
    ef = jnp.maximum(element_fractions.reshape(-1), 0.0)
    partials = _sc_charge_loss(pw, ef, oxidation_states)
    charge_balance_loss = jnp.sum(partials[:, 0, :])
    mean_charge_imbalance = jnp.sum(partials[:, 1, :])
    return (charge_balance_loss, mean_charge_imbalance)
